# baseline (speedup 1.0000x reference)
import functools

import jax
import jax.numpy as jnp
from jax import lax
from jax.experimental import pallas as pl
from jax.experimental.pallas import tpu as pltpu
from jax.experimental.pallas import tpu_sc as plsc

_B = 16384
_D = 64
_NC = 2
_NS = 16
_NW = _NC * _NS
_BPW = _B // _NW
_HALF = _BPW // 2
_CHUNK = 128
_L = 16


def _bpr_gather(user, pos_item, neg_item, user_emb2, item_emb2):
    mesh = plsc.VectorSubcoreMesh(
        core_axis_name="c", subcore_axis_name="s",
        num_cores=_NC, num_subcores=_NS)
    row = jax.ShapeDtypeStruct((_B, _D), jnp.float32)

    @functools.partial(
        pl.kernel,
        mesh=mesh,
        out_type=(row, row, row),
        compiler_params=pltpu.CompilerParams(use_tc_tiling_on_sc=True,
                                             needs_layout_passes=False),
        scratch_types=[
            pltpu.VMEM((_BPW,), jnp.int32),
            pltpu.VMEM((_BPW,), jnp.int32),
            pltpu.VMEM((_HALF, 2 * _D), jnp.float32),
            pltpu.VMEM((_BPW, _D), jnp.float32),
            pltpu.SemaphoreType.DMA,
            pltpu.SemaphoreType.DMA,
        ],
    )
    def body(user_h, pos_h, neg_h, uemb_h, iemb_h,
             u_out, i_out, j_out,
             idxv, pidx, pairs, outbuf,
             gsem, wsem):
        wid = lax.axis_index("s") * _NC + lax.axis_index("c")
        base = wid * _BPW
        iota = lax.iota(jnp.int32, _L)
        lookups = ((user_h, uemb_h, u_out),
                   (pos_h, iemb_h, i_out),
                   (neg_h, iemb_h, j_out))
        prev_write = None
        for idx_h, tbl, out in lookups:
            pltpu.sync_copy(idx_h.at[pl.ds(base, _BPW)], idxv)
            for c in range(_BPW // _L):
                sl = pl.ds(c * _L, _L)
                pidx[sl] = lax.shift_right_logical(idxv[sl], 1)
            for hh in range(2):
                gathers = []
                for c in range(_HALF // _CHUNK):
                    src_sl = pl.ds(hh * _HALF + c * _CHUNK, _CHUNK)
                    dst_sl = pl.ds(c * _CHUNK, _CHUNK)
                    gathers.append(
                        pltpu.async_copy(tbl.at[pidx.at[src_sl]],
                                         pairs.at[dst_sl, :], gsem))
                if prev_write is not None:
                    prev_write.wait()
                    prev_write = None
                for h in gathers:
                    h.wait()

                def ext(c, carry, hh=hh):
                    kl = c * _L + iota
                    kg = hh * _HALF + kl
                    par = idxv[pl.ds(hh * _HALF + c * _L, _L)] & 1
                    off = par * _D
                    for w in range(_D):
                        wv = jnp.full((_L,), w, jnp.int32)
                        v = plsc.load_gather(pairs, [kl, off + w])
                        plsc.store_scatter(outbuf, [kg, wv], v)
                    return carry

                lax.fori_loop(0, _HALF // _L, ext, 0)
            prev_write = pltpu.async_copy(
                outbuf, out.at[pl.ds(base, _BPW)], wsem)
        prev_write.wait()

    return body(user, pos_item, neg_item, user_emb2, item_emb2)


def kernel(user, pos_item, neg_item, user_emb, item_emb):
    return _bpr_gather(user, pos_item, neg_item,
                       user_emb.reshape(-1, 2 * _D),
                       item_emb.reshape(-1, 2 * _D))


# speedup vs baseline: 1.1825x; 1.1825x over previous
import functools

import jax
import jax.numpy as jnp
from jax import lax
from jax.experimental import pallas as pl
from jax.experimental.pallas import tpu as pltpu
from jax.experimental.pallas import tpu_sc as plsc

_B = 16384
_D = 64
_NC = 2
_NS = 16
_NW = _NC * _NS
_BPW = _B // _NW
_CHUNK = 128
_NCHUNKS = _BPW // _CHUNK

_MESH = plsc.VectorSubcoreMesh(
    core_axis_name="c", subcore_axis_name="s",
    num_cores=_NC, num_subcores=_NS)
_ROW = jax.ShapeDtypeStruct((_B, 2 * _D), jnp.float32)


def _gather_kernel(n_lookups):
    @functools.partial(
        pl.kernel,
        mesh=_MESH,
        out_type=(_ROW,) * n_lookups,
        compiler_params=pltpu.CompilerParams(use_tc_tiling_on_sc=True,
                                             needs_layout_passes=False),
        scratch_types=[
            *[pltpu.VMEM((_BPW,), jnp.int32) for _ in range(n_lookups)],
            *[pltpu.VMEM((_BPW // 2, 2 * _D), jnp.float32)
              for _ in range(n_lookups)],
            *[pltpu.SemaphoreType.DMA for _ in range(n_lookups)],
            pltpu.SemaphoreType.DMA,
        ],
    )
    def body(tbl, *rest):
        idx_hs = rest[:n_lookups]
        outs = rest[n_lookups:2 * n_lookups]
        idxvs = rest[2 * n_lookups:3 * n_lookups]
        rowss = rest[3 * n_lookups:4 * n_lookups]
        gsems = rest[4 * n_lookups:5 * n_lookups]
        wsem = rest[5 * n_lookups]
        wid = lax.axis_index("s") * _NC + lax.axis_index("c")
        base = wid * _BPW
        half = _BPW // 2
        for idx_h, idxv in zip(idx_hs, idxvs):
            pltpu.sync_copy(idx_h.at[pl.ds(base, _BPW)], idxv)
        writes = []
        for hh in range(2):
            gathers = [[] for _ in range(n_lookups)]
            for c in range(half // _CHUNK):
                src_sl = pl.ds(hh * half + c * _CHUNK, _CHUNK)
                dst_sl = pl.ds(c * _CHUNK, _CHUNK)
                for t in range(n_lookups):
                    gathers[t].append(
                        pltpu.async_copy(tbl.at[idxvs[t].at[src_sl]],
                                         rowss[t].at[dst_sl, :], gsems[t]))
            for t in range(n_lookups):
                for h in gathers[t]:
                    h.wait()
                writes.append(
                    pltpu.async_copy(rowss[t],
                                     outs[t].at[pl.ds(base + hh * half, half)],
                                     wsem))
            if hh == 0:
                for h in writes:
                    h.wait()
                writes = []
        for h in writes:
            h.wait()

    return body


def kernel(user, pos_item, neg_item, user_emb, item_emb):
    pad = jnp.zeros((1, 2 * _D - _D), jnp.float32)
    ue = jnp.concatenate(
        [user_emb, jnp.broadcast_to(pad, (user_emb.shape[0], _D))], axis=1)
    ie = jnp.concatenate(
        [item_emb, jnp.broadcast_to(pad, (item_emb.shape[0], _D))], axis=1)
    (u,) = _gather_kernel(1)(ue, user)
    i, j = _gather_kernel(2)(ie, pos_item, neg_item)
    return (u[:, :_D], i[:, :_D], j[:, :_D])
